# baseline (speedup 1.0000x reference)
import jax
import jax.numpy as jnp
from jax import lax
from jax.experimental import pallas as pl
from jax.experimental.pallas import tpu as pltpu
from jax.experimental.pallas import tpu_sc as plsc

BATCH = 4096
SEQ = 200
D = 64

NC = 2
NS = 16
NW = NC * NS
L = 16

BW = BATCH // NW
PBW = BW + 1
ST = SEQ // 8
NBUF = 4


def _transpose_add(gbuf, sbuf, pos_v, b, s):
    iota = lax.iota(jnp.int32, L)
    ps = [pos_v[s, pl.ds(c * L, L)] for c in range(D // L)]
    c8v = [(c * L + iota) // 8 for c in range(D // L)]
    drv = [(c * L + iota) % 8 for c in range(D // L)]

    @plsc.parallel_loop(0, BW, unroll=8)
    def body(r):
        rv = jnp.full((L,), 0, jnp.int32) + r
        for c in range(D // L):
            y = gbuf[b, r, pl.ds(c * L, L)] + ps[c]
            plsc.store_scatter(sbuf.at[b], [c8v[c], drv[c], rv], y)


def _kernel_body(ids_hbm, tok_hbm, pos_hbm, out_hbm,
                 idx_v, pos_v, gbuf, sbuf, gs0, gs1, gs2, gs3,
                 ss0, ss1, ss2, ss3):
    gsem = (gs0, gs1, gs2, gs3)
    ssem = (ss0, ss1, ss2, ss3)
    wid = lax.axis_index("s") * NC + lax.axis_index("c")

    pltpu.sync_copy(ids_hbm.at[:, wid], idx_v)
    pltpu.sync_copy(pos_hbm, pos_v)

    def start_gather(s, b):
        pltpu.make_async_copy(
            tok_hbm.at[idx_v.at[lax.div(s, 8), lax.rem(s, 8)]],
            gbuf.at[b], gsem[b]).start()

    def wait_gather(b):
        pltpu.make_async_copy(
            tok_hbm.at[idx_v.at[0, 0]], gbuf.at[b], gsem[b]).wait()

    def start_store(s, b):
        for c8 in range(D // 8):
            pltpu.make_async_copy(
                sbuf.at[b, c8, :, pl.ds(0, BW)],
                out_hbm.at[s, c8, wid], ssem[b]).start()

    def wait_store(b):
        for c8 in range(D // 8):
            pltpu.make_async_copy(
                sbuf.at[b, c8, :, pl.ds(0, BW)],
                out_hbm.at[0, c8, wid], ssem[b]).wait()

    start_gather(0, 0)
    start_gather(1, 1)
    for s in (0, 1):
        start_gather(s + 2, s + 2)
        wait_gather(s)
        _transpose_add(gbuf, sbuf, pos_v, s, s)
        start_store(s, s)

    def group(g, carry):
        for j in range(NBUF):
            s = 2 + g * NBUF + j
            b = (2 + j) % NBUF
            wait_store(j)
            start_gather(s + 2, j)
            wait_gather(b)
            _transpose_add(gbuf, sbuf, pos_v, b, s)
            start_store(s, b)
        return carry

    lax.fori_loop(0, (SEQ - NBUF) // NBUF, group, 0)

    for s in (SEQ - 2, SEQ - 1):
        b = s % NBUF
        wait_store(b - 2)
        wait_gather(b)
        _transpose_add(gbuf, sbuf, pos_v, b, s)
        start_store(s, b)
    wait_store(NBUF - 2)
    wait_store(NBUF - 1)


def kernel(input_ids, token_table, pos_table):
    ids_phys = (input_ids.astype(jnp.int32).T
                .reshape(ST, 8, NW, BW).transpose(0, 2, 1, 3))
    tok = token_table.astype(jnp.float32)
    pos = pos_table.astype(jnp.float32)

    mesh = plsc.VectorSubcoreMesh(core_axis_name="c", subcore_axis_name="s")
    run = pl.kernel(
        _kernel_body,
        out_type=jax.ShapeDtypeStruct((SEQ, D // 8, NW, 8, BW), jnp.float32),
        mesh=mesh,
        compiler_params=pltpu.CompilerParams(
            use_tc_tiling_on_sc=False, needs_layout_passes=False),
        scratch_types=[
            pltpu.VMEM((ST, 8, BW), jnp.int32),
            pltpu.VMEM((SEQ, D), jnp.float32),
            pltpu.VMEM((NBUF, BW, D), jnp.float32),
            pltpu.VMEM((NBUF, D // 8, 8, PBW), jnp.float32),
            pltpu.SemaphoreType.DMA, pltpu.SemaphoreType.DMA,
            pltpu.SemaphoreType.DMA, pltpu.SemaphoreType.DMA,
            pltpu.SemaphoreType.DMA, pltpu.SemaphoreType.DMA,
            pltpu.SemaphoreType.DMA, pltpu.SemaphoreType.DMA,
        ],
    )
    out5 = run(ids_phys, tok, pos)
    return out5.transpose(2, 4, 0, 1, 3).reshape(BATCH, SEQ, D)


# speedup vs baseline: 1.2577x; 1.2577x over previous
import jax
import jax.numpy as jnp
from jax import lax
from jax.experimental import pallas as pl
from jax.experimental.pallas import tpu as pltpu
from jax.experimental.pallas import tpu_sc as plsc

BATCH = 4096
SEQ = 200
D = 64

NC = 2
NS = 16
NW = NC * NS
L = 16

BW = BATCH // NW
PBW = BW + 1
ST = SEQ // 8
NBUF = 4


def _transpose_add(gbuf, sbuf, pos_v, b, s):
    iota = lax.iota(jnp.int32, L)
    ps = [pos_v[s, pl.ds(c * L, L)] for c in range(D // L)]
    c8v = [(c * L + iota) // 8 for c in range(D // L)]
    drv = [(c * L + iota) % 8 for c in range(D // L)]

    @plsc.parallel_loop(0, BW, unroll=2)
    def body(r):
        rv = jnp.full((L,), 0, jnp.int32) + r
        for c in range(D // L):
            y = gbuf[b, r, pl.ds(c * L, L)] + ps[c]
            plsc.store_scatter(sbuf.at[b], [c8v[c], drv[c], rv], y)


def _kernel_body(ids_hbm, tok_hbm, pos_hbm, out_hbm,
                 idx_v, pos_v, gbuf, sbuf, gs0, gs1, gs2, gs3,
                 ss0, ss1, ss2, ss3):
    gsem = (gs0, gs1, gs2, gs3)
    ssem = (ss0, ss1, ss2, ss3)
    wid = lax.axis_index("s") * NC + lax.axis_index("c")

    pltpu.sync_copy(ids_hbm.at[:, wid], idx_v)
    pltpu.sync_copy(pos_hbm, pos_v)

    def start_gather(s, b):
        pltpu.make_async_copy(
            tok_hbm.at[idx_v.at[lax.div(s, 8), lax.rem(s, 8)]],
            gbuf.at[b], gsem[b]).start()

    def wait_gather(b):
        pltpu.make_async_copy(
            tok_hbm.at[idx_v.at[0, 0]], gbuf.at[b], gsem[b]).wait()

    def start_store(s, b):
        for c8 in range(D // 8):
            pltpu.make_async_copy(
                sbuf.at[b, c8, :, pl.ds(0, BW)],
                out_hbm.at[s, c8, wid], ssem[b]).start()

    def wait_store(b):
        for c8 in range(D // 8):
            pltpu.make_async_copy(
                sbuf.at[b, c8, :, pl.ds(0, BW)],
                out_hbm.at[0, c8, wid], ssem[b]).wait()

    start_gather(0, 0)
    start_gather(1, 1)
    for s in (0, 1):
        start_gather(s + 2, s + 2)
        wait_gather(s)
        _transpose_add(gbuf, sbuf, pos_v, s, s)
        start_store(s, s)

    def group(g, carry):
        for j in range(NBUF):
            s = 2 + g * NBUF + j
            b = (2 + j) % NBUF
            wait_store(j)
            start_gather(s + 2, j)
            wait_gather(b)
            _transpose_add(gbuf, sbuf, pos_v, b, s)
            start_store(s, b)
        return carry

    lax.fori_loop(0, (SEQ - NBUF) // NBUF, group, 0)

    for s in (SEQ - 2, SEQ - 1):
        b = s % NBUF
        wait_store(b - 2)
        wait_gather(b)
        _transpose_add(gbuf, sbuf, pos_v, b, s)
        start_store(s, b)
    wait_store(NBUF - 2)
    wait_store(NBUF - 1)


def kernel(input_ids, token_table, pos_table):
    ids_phys = (input_ids.astype(jnp.int32).T
                .reshape(ST, 8, NW, BW).transpose(0, 2, 1, 3))
    tok = token_table.astype(jnp.float32)
    pos = pos_table.astype(jnp.float32)

    mesh = plsc.VectorSubcoreMesh(core_axis_name="c", subcore_axis_name="s")
    run = pl.kernel(
        _kernel_body,
        out_type=jax.ShapeDtypeStruct((SEQ, D // 8, NW, 8, BW), jnp.float32),
        mesh=mesh,
        compiler_params=pltpu.CompilerParams(
            use_tc_tiling_on_sc=False, needs_layout_passes=False),
        scratch_types=[
            pltpu.VMEM((ST, 8, BW), jnp.int32),
            pltpu.VMEM((SEQ, D), jnp.float32),
            pltpu.VMEM((NBUF, BW, D), jnp.float32),
            pltpu.VMEM((NBUF, D // 8, 8, PBW), jnp.float32),
            pltpu.SemaphoreType.DMA, pltpu.SemaphoreType.DMA,
            pltpu.SemaphoreType.DMA, pltpu.SemaphoreType.DMA,
            pltpu.SemaphoreType.DMA, pltpu.SemaphoreType.DMA,
            pltpu.SemaphoreType.DMA, pltpu.SemaphoreType.DMA,
        ],
    )
    out5 = run(ids_phys, tok, pos)
    return out5.transpose(2, 4, 0, 1, 3).reshape(BATCH, SEQ, D)


# speedup vs baseline: 1.2639x; 1.0049x over previous
import jax
import jax.numpy as jnp
from jax import lax
from jax.experimental import pallas as pl
from jax.experimental.pallas import tpu as pltpu
from jax.experimental.pallas import tpu_sc as plsc

BATCH = 4096
SEQ = 200
D = 64

NC = 2
NS = 16
NW = NC * NS
L = 16

BW = BATCH // NW
PBW = BW + 1
ST = SEQ // 8
NBUF = 4


def _transpose_add(gbuf, sbuf, pos_v, b, s):
    iota = lax.iota(jnp.int32, L)
    ps = [pos_v[s, pl.ds(c * L, L)] for c in range(D // L)]
    c8v = [(c * L + iota) // 8 for c in range(D // L)]
    drv = [(c * L + iota) % 8 for c in range(D // L)]

    @plsc.parallel_loop(0, BW, unroll=2)
    def body(r):
        rv = jnp.full((L,), 0, jnp.int32) + r
        for c in range(D // L):
            y = gbuf[b, r, pl.ds(c * L, L)] + ps[c]
            plsc.store_scatter(sbuf.at[b], [c8v[c], drv[c], rv], y)


def _kernel_body(ids_hbm, tok_hbm, pos_hbm, out_hbm,
                 idx_v, pos_v, gbuf, sbuf, gs0, gs1, gs2, gs3,
                 ss0, ss1, ss2, ss3):
    gsem = (gs0, gs1, gs2, gs3)
    ssem = (ss0, ss1, ss2, ss3)
    wid = lax.axis_index("s") * NC + lax.axis_index("c")

    pltpu.sync_copy(ids_hbm.at[:, wid], idx_v)
    pltpu.sync_copy(pos_hbm, pos_v)

    def start_gather(s, b):
        pltpu.make_async_copy(
            tok_hbm.at[idx_v.at[lax.div(s, 8), lax.rem(s, 8)]],
            gbuf.at[b], gsem[b]).start()

    def wait_gather(b):
        pltpu.make_async_copy(
            tok_hbm.at[idx_v.at[0, 0]], gbuf.at[b], gsem[b]).wait()

    def start_store(s, b):
        pltpu.make_async_copy(
            sbuf.at[b, :, :, pl.ds(0, BW)],
            out_hbm.at[s, :, wid], ssem[b]).start()

    def wait_store(b):
        pltpu.make_async_copy(
            sbuf.at[b, :, :, pl.ds(0, BW)],
            out_hbm.at[0, :, wid], ssem[b]).wait()

    start_gather(0, 0)
    start_gather(1, 1)
    for s in (0, 1):
        start_gather(s + 2, s + 2)
        wait_gather(s)
        _transpose_add(gbuf, sbuf, pos_v, s, s)
        start_store(s, s)

    def group(g, carry):
        for j in range(NBUF):
            s = 2 + g * NBUF + j
            b = (2 + j) % NBUF
            wait_store(j)
            start_gather(s + 2, j)
            wait_gather(b)
            _transpose_add(gbuf, sbuf, pos_v, b, s)
            start_store(s, b)
        return carry

    lax.fori_loop(0, (SEQ - NBUF) // NBUF, group, 0)

    for s in (SEQ - 2, SEQ - 1):
        b = s % NBUF
        wait_store(b - 2)
        wait_gather(b)
        _transpose_add(gbuf, sbuf, pos_v, b, s)
        start_store(s, b)
    wait_store(NBUF - 2)
    wait_store(NBUF - 1)


def kernel(input_ids, token_table, pos_table):
    ids_phys = (input_ids.astype(jnp.int32).T
                .reshape(ST, 8, NW, BW).transpose(0, 2, 1, 3))
    tok = token_table.astype(jnp.float32)
    pos = pos_table.astype(jnp.float32)

    mesh = plsc.VectorSubcoreMesh(core_axis_name="c", subcore_axis_name="s")
    run = pl.kernel(
        _kernel_body,
        out_type=jax.ShapeDtypeStruct((SEQ, D // 8, NW, 8, BW), jnp.float32),
        mesh=mesh,
        compiler_params=pltpu.CompilerParams(
            use_tc_tiling_on_sc=False, needs_layout_passes=False),
        scratch_types=[
            pltpu.VMEM((ST, 8, BW), jnp.int32),
            pltpu.VMEM((SEQ, D), jnp.float32),
            pltpu.VMEM((NBUF, BW, D), jnp.float32),
            pltpu.VMEM((NBUF, D // 8, 8, PBW), jnp.float32),
            pltpu.SemaphoreType.DMA, pltpu.SemaphoreType.DMA,
            pltpu.SemaphoreType.DMA, pltpu.SemaphoreType.DMA,
            pltpu.SemaphoreType.DMA, pltpu.SemaphoreType.DMA,
            pltpu.SemaphoreType.DMA, pltpu.SemaphoreType.DMA,
        ],
    )
    out5 = run(ids_phys, tok, pos)
    return out5.transpose(2, 4, 0, 1, 3).reshape(BATCH, SEQ, D)


# speedup vs baseline: 1.2942x; 1.0240x over previous
import jax
import jax.numpy as jnp
from jax import lax
from jax.experimental import pallas as pl
from jax.experimental.pallas import tpu as pltpu
from jax.experimental.pallas import tpu_sc as plsc

BATCH = 4096
SEQ = 200
D = 64

NC = 2
NS = 16
NW = NC * NS
L = 16

BW = BATCH // NW
PBW = BW + 1
ST = SEQ // 8
NBUF = 5


def _transpose_add(gbuf, sbuf, pos_v, b, s):
    iota = lax.iota(jnp.int32, L)
    ps = [pos_v[s, pl.ds(c * L, L)] for c in range(D // L)]
    c8v = [(c * L + iota) // 8 for c in range(D // L)]
    drv = [(c * L + iota) % 8 for c in range(D // L)]

    @plsc.parallel_loop(0, BW, unroll=2)
    def body(r):
        rv = jnp.full((L,), 0, jnp.int32) + r
        for c in range(D // L):
            y = gbuf[b, r, pl.ds(c * L, L)] + ps[c]
            plsc.store_scatter(sbuf.at[b], [c8v[c], drv[c], rv], y)


def _kernel_body(ids_hbm, tok_hbm, pos_hbm, out_hbm,
                 idx_v, pos_v, gbuf, sbuf, gs0, gs1, gs2, gs3, gs4,
                 ss0, ss1, ss2, ss3, ss4):
    gsem = (gs0, gs1, gs2, gs3, gs4)
    ssem = (ss0, ss1, ss2, ss3, ss4)
    wid = lax.axis_index("s") * NC + lax.axis_index("c")

    pltpu.sync_copy(ids_hbm.at[:, wid], idx_v)
    pltpu.sync_copy(pos_hbm, pos_v)

    def start_gather(s, b):
        pltpu.make_async_copy(
            tok_hbm.at[idx_v.at[lax.div(s, 8), lax.rem(s, 8)]],
            gbuf.at[b], gsem[b]).start()

    def wait_gather(b):
        pltpu.make_async_copy(
            tok_hbm.at[idx_v.at[0, 0]], gbuf.at[b], gsem[b]).wait()

    def start_store(s, b):
        pltpu.make_async_copy(
            sbuf.at[b, :, :, pl.ds(0, BW)],
            out_hbm.at[s, :, wid], ssem[b]).start()

    def wait_store(b):
        pltpu.make_async_copy(
            sbuf.at[b, :, :, pl.ds(0, BW)],
            out_hbm.at[0, :, wid], ssem[b]).wait()

    for g0 in range(3):
        start_gather(g0, g0)
    for s in (0, 1):
        start_gather(s + 3, s + 3)
        wait_gather(s)
        _transpose_add(gbuf, sbuf, pos_v, s, s)
        start_store(s, s)

    def group(g, carry):
        for j in range(NBUF):
            s = 2 + g * NBUF + j
            b = (2 + j) % NBUF
            wait_store(j)
            start_gather(s + 3, j)
            wait_gather(b)
            _transpose_add(gbuf, sbuf, pos_v, b, s)
            start_store(s, b)
        return carry

    lax.fori_loop(0, (SEQ - NBUF) // NBUF, group, 0)

    for s in (SEQ - 3, SEQ - 2, SEQ - 1):
        b = s % NBUF
        wait_store((s + 3) % NBUF)
        wait_gather(b)
        _transpose_add(gbuf, sbuf, pos_v, b, s)
        start_store(s, b)
    wait_store((SEQ - 2) % NBUF)
    wait_store((SEQ - 1) % NBUF)


def kernel(input_ids, token_table, pos_table):
    ids_phys = (input_ids.astype(jnp.int32).T
                .reshape(ST, 8, NW, BW).transpose(0, 2, 1, 3))
    tok = token_table.astype(jnp.float32)
    pos = pos_table.astype(jnp.float32)

    mesh = plsc.VectorSubcoreMesh(core_axis_name="c", subcore_axis_name="s")
    run = pl.kernel(
        _kernel_body,
        out_type=jax.ShapeDtypeStruct((SEQ, D // 8, NW, 8, BW), jnp.float32),
        mesh=mesh,
        compiler_params=pltpu.CompilerParams(
            use_tc_tiling_on_sc=False, needs_layout_passes=False),
        scratch_types=[
            pltpu.VMEM((ST, 8, BW), jnp.int32),
            pltpu.VMEM((SEQ, D), jnp.float32),
            pltpu.VMEM((NBUF, BW, D), jnp.float32),
            pltpu.VMEM((NBUF, D // 8, 8, PBW), jnp.float32),
            pltpu.SemaphoreType.DMA, pltpu.SemaphoreType.DMA,
            pltpu.SemaphoreType.DMA, pltpu.SemaphoreType.DMA,
            pltpu.SemaphoreType.DMA, pltpu.SemaphoreType.DMA,
            pltpu.SemaphoreType.DMA, pltpu.SemaphoreType.DMA,
            pltpu.SemaphoreType.DMA, pltpu.SemaphoreType.DMA,
        ],
    )
    out5 = run(ids_phys, tok, pos)
    return out5.transpose(2, 4, 0, 1, 3).reshape(BATCH, SEQ, D)
